# Initial kernel scaffold; baseline (speedup 1.0000x reference)
import jax
import jax.numpy as jnp
from jax.experimental import pallas as pl


def kernel(x, table, W1, b1, W2, b2):
    raise NotImplementedError("write your pallas kernel here")


# speedup vs baseline: 4.5771x; 4.5771x over previous
import functools

import jax
import jax.numpy as jnp
from jax import lax
from jax.experimental import pallas as pl
from jax.experimental.pallas import tpu as pltpu
from jax.experimental.pallas import tpu_sc as plsc

VOCAB = 100000
EMBED = 128
HIDDEN = 1024
OUT = 256
BATCH = 16384
SEQ = 20

NC = 2
NS = 16
L = 16
NW = NC * NS
BPW = BATCH // NW

CB = 32
IDX_PER_CHUNK = CB * SEQ
NG = IDX_PER_CHUNK // 128
NCHUNK = BPW // CB
EV = EMBED // L


def _sc_pool(x3, table):
    mesh = plsc.VectorSubcoreMesh(
        core_axis_name="c", subcore_axis_name="s", num_cores=NC,
        num_subcores=NS)

    @functools.partial(
        pl.kernel,
        out_type=jax.ShapeDtypeStruct((BATCH, EMBED), jnp.float32),
        mesh=mesh,
        scratch_types=[
            pltpu.VMEM((NG, 128), jnp.int32),
            pltpu.VMEM((IDX_PER_CHUNK, EMBED), jnp.float32),
            pltpu.VMEM((CB, EMBED), jnp.float32),
            pltpu.SemaphoreType.DMA,
        ],
    )
    def k(x_hbm, table_hbm, out_hbm, idx_v, rows_v, pooled_v, sem):
        wid = lax.axis_index("s") * NC + lax.axis_index("c")

        def chunk_body(g, carry):
            pltpu.sync_copy(x_hbm.at[wid * NCHUNK + g], idx_v)
            copies = [
                pltpu.async_copy(
                    table_hbm.at[idx_v.at[j]],
                    rows_v.at[pl.ds(j * 128, 128)],
                    sem,
                )
                for j in range(NG)
            ]
            for c in copies:
                c.wait()

            def pool_body(b, carry2):
                r0 = b * SEQ
                for d in range(EV):
                    col = pl.ds(d * L, L)
                    acc = rows_v[r0, col]
                    for j in range(1, SEQ):
                        acc = acc + rows_v[r0 + j, col]
                    pooled_v[b, col] = 1.0 / (1.0 + jnp.exp(-acc))
                return carry2

            lax.fori_loop(0, CB, pool_body, 0)
            b0 = wid * BPW + g * CB
            pltpu.sync_copy(pooled_v, out_hbm.at[pl.ds(b0, CB)])
            return carry

        lax.fori_loop(0, NCHUNK, chunk_body, 0)

    return k(x3, table)


def _mlp(s, W1, b1, W2, b2):
    BB = 1024

    def body(s_ref, w1_ref, b1_ref, w2_ref, b2_ref, o_ref):
        h = jnp.dot(s_ref[...], w1_ref[...],
                    preferred_element_type=jnp.float32) + b1_ref[...]
        h = jax.nn.sigmoid(h)
        o_ref[...] = jnp.dot(h, w2_ref[...],
                             preferred_element_type=jnp.float32) + b2_ref[...]

    return pl.pallas_call(
        body,
        grid=(BATCH // BB,),
        in_specs=[
            pl.BlockSpec((BB, EMBED), lambda i: (i, 0)),
            pl.BlockSpec((EMBED, HIDDEN), lambda i: (0, 0)),
            pl.BlockSpec((1, HIDDEN), lambda i: (0, 0)),
            pl.BlockSpec((HIDDEN, OUT), lambda i: (0, 0)),
            pl.BlockSpec((1, OUT), lambda i: (0, 0)),
        ],
        out_specs=pl.BlockSpec((BB, OUT), lambda i: (i, 0)),
        out_shape=jax.ShapeDtypeStruct((BATCH, OUT), jnp.float32),
    )(s, W1, b1.reshape(1, HIDDEN), W2, b2.reshape(1, OUT))


def kernel(x, table, W1, b1, W2, b2):
    x3 = x.reshape(NW * NCHUNK, NG, 128)
    s = _sc_pool(x3, table)
    return _mlp(s, W1, b1, W2, b2)


# speedup vs baseline: 8.1428x; 1.7790x over previous
import functools

import jax
import jax.numpy as jnp
from jax import lax
from jax.experimental import pallas as pl
from jax.experimental.pallas import tpu as pltpu
from jax.experimental.pallas import tpu_sc as plsc

VOCAB = 100000
EMBED = 128
HIDDEN = 1024
OUT = 256
BATCH = 16384
SEQ = 20

NC = 2
NS = 16
L = 16
NW = NC * NS
BPW = BATCH // NW
IPW = BPW * SEQ

SEGB = 8
SEG = SEGB * SEQ
RING = 4
NSEG = BPW // SEGB
EV = EMBED // L


def _sc_pool(x_flat, table):
    mesh = plsc.VectorSubcoreMesh(
        core_axis_name="c", subcore_axis_name="s", num_cores=NC,
        num_subcores=NS)

    @functools.partial(
        pl.kernel,
        out_type=jax.ShapeDtypeStruct((BATCH, EMBED), jnp.float32),
        mesh=mesh,
        scratch_types=[
            pltpu.VMEM((IPW,), jnp.int32),
            pltpu.VMEM((RING * SEG, EMBED), jnp.float32),
            pltpu.VMEM((RING * SEGB, EMBED), jnp.float32),
            pltpu.SemaphoreType.DMA,
            pltpu.SemaphoreType.DMA,
        ],
    )
    def k(x_hbm, table_hbm, out_hbm, idx_v, rows_v, pooled_v, gsem, osem):
        wid = lax.axis_index("s") * NC + lax.axis_index("c")

        pltpu.sync_copy(
            x_hbm.at[pl.ds(pl.multiple_of(wid * IPW, 8), IPW)], idx_v)

        def fire_gather(s):
            start = pl.multiple_of(s * SEG, 8)
            slot = pl.multiple_of(lax.rem(s, RING) * SEG, 8)
            pltpu.async_copy(
                table_hbm.at[idx_v.at[pl.ds(start, SEG)]],
                rows_v.at[pl.ds(slot, SEG)],
                gsem,
            )

        for r in range(RING):
            fire_gather(r)

        def seg_body(s, carry):
            slot = lax.rem(s, RING)
            rbase = pl.multiple_of(slot * SEG, 8)
            pbase = pl.multiple_of(slot * SEGB, 8)
            pltpu.make_async_copy(
                table_hbm.at[pl.ds(0, SEG)],
                rows_v.at[pl.ds(rbase, SEG)],
                gsem,
            ).wait()
            @pl.when(s >= RING)
            def _():
                pltpu.make_async_copy(
                    pooled_v.at[pl.ds(pbase, SEGB)],
                    out_hbm.at[pl.ds(0, SEGB)],
                    osem,
                ).wait()

            def pool_body(b, carry2):
                r0 = rbase + b * SEQ
                for d in range(EV):
                    col = pl.ds(d * L, L)
                    acc = rows_v[r0, col]
                    for j in range(1, SEQ):
                        acc = acc + rows_v[r0 + j, col]
                    pooled_v[pbase + b, col] = acc
                return carry2

            lax.fori_loop(0, SEGB, pool_body, 0)
            pltpu.async_copy(
                pooled_v.at[pl.ds(pbase, SEGB)],
                out_hbm.at[pl.ds(
                    pl.multiple_of(wid * BPW + s * SEGB, 8), SEGB)],
                osem,
            )

            @pl.when(s < NSEG - RING)
            def _():
                fire_gather(s + RING)

            return carry

        lax.fori_loop(0, NSEG, seg_body, 0)

        for r in range(RING):
            pltpu.make_async_copy(
                pooled_v.at[pl.ds(r * SEGB, SEGB)],
                out_hbm.at[pl.ds(0, SEGB)],
                osem,
            ).wait()

    return k(x_flat, table)


def _mlp(s, W1, b1, W2, b2):
    BB = 1024

    def body(s_ref, w1_ref, b1_ref, w2_ref, b2_ref, o_ref):
        sv = jax.nn.sigmoid(s_ref[...])
        h = jnp.dot(sv, w1_ref[...],
                    preferred_element_type=jnp.float32) + b1_ref[...]
        h = jax.nn.sigmoid(h)
        o_ref[...] = jnp.dot(h, w2_ref[...],
                             preferred_element_type=jnp.float32) + b2_ref[...]

    return pl.pallas_call(
        body,
        grid=(BATCH // BB,),
        in_specs=[
            pl.BlockSpec((BB, EMBED), lambda i: (i, 0)),
            pl.BlockSpec((EMBED, HIDDEN), lambda i: (0, 0)),
            pl.BlockSpec((1, HIDDEN), lambda i: (0, 0)),
            pl.BlockSpec((HIDDEN, OUT), lambda i: (0, 0)),
            pl.BlockSpec((1, OUT), lambda i: (0, 0)),
        ],
        out_specs=pl.BlockSpec((BB, OUT), lambda i: (i, 0)),
        out_shape=jax.ShapeDtypeStruct((BATCH, OUT), jnp.float32),
    )(s, W1, b1.reshape(1, HIDDEN), W2, b2.reshape(1, OUT))


def kernel(x, table, W1, b1, W2, b2):
    s = _sc_pool(x.reshape(BATCH * SEQ), table)
    return _mlp(s, W1, b1, W2, b2)


# speedup vs baseline: 8.1521x; 1.0011x over previous
import functools

import jax
import jax.numpy as jnp
from jax import lax
from jax.experimental import pallas as pl
from jax.experimental.pallas import tpu as pltpu
from jax.experimental.pallas import tpu_sc as plsc

VOCAB = 100000
EMBED = 128
HIDDEN = 1024
OUT = 256
BATCH = 16384
SEQ = 20

NC = 2
NS = 16
L = 16
NW = NC * NS
BPW = BATCH // NW
IPW = BPW * SEQ

SEGB = 8
SEG = SEGB * SEQ
RING = 4
NSEG = BPW // SEGB
EV = EMBED // L


def _sc_pool(x_flat, table):
    mesh = plsc.VectorSubcoreMesh(
        core_axis_name="c", subcore_axis_name="s", num_cores=NC,
        num_subcores=NS)

    @functools.partial(
        pl.kernel,
        out_type=jax.ShapeDtypeStruct((BATCH, EMBED), jnp.float32),
        mesh=mesh,
        scratch_types=[
            pltpu.VMEM((IPW,), jnp.int32),
            pltpu.VMEM((RING * SEG, EMBED), jnp.float32),
            pltpu.VMEM((RING * SEGB, EMBED), jnp.float32),
            pltpu.SemaphoreType.DMA,
            pltpu.SemaphoreType.DMA,
        ],
        compiler_params=pltpu.CompilerParams(needs_layout_passes=False),
    )
    def k(x_hbm, table_hbm, out_hbm, idx_v, rows_v, pooled_v, gsem, osem):
        wid = lax.axis_index("s") * NC + lax.axis_index("c")

        pltpu.sync_copy(
            x_hbm.at[pl.ds(pl.multiple_of(wid * IPW, 8), IPW)], idx_v)

        def fire_gather(s):
            start = pl.multiple_of(s * SEG, 8)
            slot = pl.multiple_of(lax.rem(s, RING) * SEG, 8)
            pltpu.async_copy(
                table_hbm.at[idx_v.at[pl.ds(start, SEG)]],
                rows_v.at[pl.ds(slot, SEG)],
                gsem,
            )

        for r in range(RING):
            fire_gather(r)

        def seg_body(s, carry):
            slot = lax.rem(s, RING)
            rbase = pl.multiple_of(slot * SEG, 8)
            pbase = pl.multiple_of(lax.rem(s, RING) * SEGB, 8)
            pltpu.make_async_copy(
                table_hbm.at[pl.ds(0, SEG)],
                rows_v.at[pl.ds(rbase, SEG)],
                gsem,
            ).wait()
            @pl.when(s >= RING)
            def _():
                pltpu.make_async_copy(
                    pooled_v.at[pl.ds(pbase, SEGB)],
                    out_hbm.at[pl.ds(0, SEGB)],
                    osem,
                ).wait()

            def pool_body(b, carry2):
                r0 = rbase + b * SEQ
                for d in range(EV):
                    col = pl.ds(d * L, L)
                    acc = rows_v[r0, col]
                    for j in range(1, SEQ):
                        acc = acc + rows_v[r0 + j, col]
                    pooled_v[pbase + b, col] = acc
                return carry2

            lax.fori_loop(0, SEGB, pool_body, 0)
            pltpu.async_copy(
                pooled_v.at[pl.ds(pbase, SEGB)],
                out_hbm.at[pl.ds(
                    pl.multiple_of(wid * BPW + s * SEGB, 8), SEGB)],
                osem,
            )

            @pl.when(s < NSEG - RING)
            def _():
                fire_gather(s + RING)

            return carry

        lax.fori_loop(0, NSEG, seg_body, 0)

        for r in range(RING):
            pltpu.make_async_copy(
                pooled_v.at[pl.ds(r * SEGB, SEGB)],
                out_hbm.at[pl.ds(0, SEGB)],
                osem,
            ).wait()

    return k(x_flat, table)


def _mlp(s, W1p, b1, W2, b2):
    BB = 1024

    def body(s_ref, w1_ref, b1_ref, w2_ref, b2_ref, o_ref):
        sv = jax.nn.sigmoid(s_ref[...]).astype(jnp.bfloat16)
        h = jnp.dot(sv, w1_ref[...],
                    preferred_element_type=jnp.float32) + b1_ref[...]
        h = jax.nn.sigmoid(h).astype(jnp.bfloat16)
        o_ref[...] = jnp.dot(h, w2_ref[...],
                             preferred_element_type=jnp.float32) + b2_ref[...]

    return pl.pallas_call(
        body,
        grid=(BATCH // BB,),
        in_specs=[
            pl.BlockSpec((BB, EMBED), lambda i: (i, 0)),
            pl.BlockSpec((EMBED, HIDDEN), lambda i: (0, 0)),
            pl.BlockSpec((1, HIDDEN), lambda i: (0, 0)),
            pl.BlockSpec((HIDDEN, OUT), lambda i: (0, 0)),
            pl.BlockSpec((1, OUT), lambda i: (0, 0)),
        ],
        out_specs=pl.BlockSpec((BB, OUT), lambda i: (i, 0)),
        out_shape=jax.ShapeDtypeStruct((BATCH, OUT), jnp.float32),
    )(s, W1p, b1.reshape(1, HIDDEN), W2, b2.reshape(1, OUT))


def kernel(x, table, W1, b1, W2, b2):
    W1b = W1.astype(jnp.bfloat16)
    W2b = W2.astype(jnp.bfloat16)
    s = _sc_pool(x.reshape(BATCH * SEQ), table)
    return _mlp(s, W1b, b1, W2b, b2)
